# baseline (speedup 1.0000x reference)
import functools

import jax
import jax.numpy as jnp
from jax import lax
from jax.experimental import pallas as pl
from jax.experimental.pallas import tpu as pltpu
from jax.experimental.pallas import tpu_sc as plsc

_N = 8388608
_NTAB = 2000
_PAD = 2048
_L = 16
_NC, _NS = 2, 16
_NW = _NC * _NS
_PER_W = _N // _NW
_CH = 16384
_NCH = _PER_W // _CH
_VEC = 4


def _body(x_hbm, means_hbm, weights_hbm, out_hbm,
          means16_v, pair16_v, xb0, xb1, ob0, ob1,
          sem_in0, sem_in1, sem_out0, sem_out1):
  wid = lax.axis_index("s") * _NC + lax.axis_index("c")
  base = wid * _PER_W

  means_v = ob1
  cw_v = ob0
  pltpu.sync_copy(means_hbm, means_v.at[pl.ds(0, _NTAB)])
  pltpu.sync_copy(weights_hbm, cw_v.at[pl.ds(0, _NTAB)])
  for j in range(_NTAB, _PAD, _L):
    means_v[pl.ds(j, _L)] = jnp.full((_L,), jnp.inf, jnp.float32)

  def cw_step(j, carry):
    v = cw_v[pl.ds(j * _L, _L)]
    cw_v[pl.ds(j * _L, _L)] = plsc.cumsum(v) - v * 0.5 + carry
    return carry + jnp.sum(v)

  total_w = lax.fori_loop(0, _NTAB // _L, cw_step, jnp.float32(0.0))
  inv_w = jnp.ones((_L,), jnp.float32) / jnp.broadcast_to(total_w, (_L,))
  m0 = means_v[pl.ds(0, _L)][0]
  mn = means_v[pl.ds(_NTAB - _L, _L)][_L - 1]

  def scale_step(j, _):
    cw_v[pl.ds(j * _L, _L)] = cw_v[pl.ds(j * _L, _L)] * inv_w
    return _

  lax.fori_loop(0, _NTAB // _L, scale_step, jnp.float32(0.0))
  cw_v[pl.ds(_NTAB, _L)] = jnp.ones((_L,), jnp.float32)

  def rep_step(jj, _):
    v = means_v[pl.ds(jj * _L, _L)]
    for k in range(_L):
      means16_v[pl.ds((jj * _L + k + 1) * _L, _L)] = jnp.broadcast_to(v[k], (_L,))
    return _

  lax.fori_loop(0, _PAD // _L, rep_step, jnp.float32(0.0))

  def tab_step(j, _):
    a = cw_v[pl.ds(j * _L, _L)]
    b = cw_v[pl.ds(j * _L + 1, _L)]
    ma = means_v[pl.ds(j * _L, _L)]
    mb = means_v[pl.ds(j * _L + 1, _L)]
    gap = mb - ma
    sl = jnp.where(gap > 0.0, (b - a) / jnp.where(gap > 0.0, gap, 1.0), 0.0)
    bb = lax.bitcast_convert_type(b, jnp.uint32)
    bs = lax.bitcast_convert_type(sl, jnp.uint32)
    word = lax.bitcast_convert_type(
        ((bb + 0x8000) & jnp.uint32(0xFFFF0000)) | ((bs + 0x8000) >> 16),
        jnp.int32)
    for k in range(_L):
      pair16_v[pl.ds((j * _L + k + 1) * _L, _L)] = jnp.broadcast_to(word[k], (_L,))
    return _

  lax.fori_loop(0, _NTAB // _L, tab_step, jnp.float32(0.0))

  lane = lax.iota(jnp.int32, _L)
  lane_hi = lane + _L * 1024
  lob = lane + _L
  hib = lane + _L * (_NTAB - 1)
  b1023 = jnp.broadcast_to(means_v[pl.ds(1008, _L)][_L - 1], (_L,))
  b511 = jnp.broadcast_to(means_v[pl.ds(496, _L)][_L - 1], (_L,))
  b1535 = jnp.broadcast_to(means_v[pl.ds(1520, _L)][_L - 1], (_L,))
  s128 = plsc.load_gather(means_v, [lane * 128 + 127])
  s64h = plsc.load_gather(means_v, [lane * 128 + 63])

  xbufs = (xb0, xb1)
  obufs = (ob0, ob1)
  sems_in = (sem_in0, sem_in1)
  sems_out = (sem_out0, sem_out1)

  def in_copy(g, b):
    return pltpu.make_async_copy(
        x_hbm.at[pl.ds(base + g * _CH, _CH)], xbufs[b], sems_in[b])

  def out_copy(g, b):
    return pltpu.make_async_copy(
        obufs[b], out_hbm.at[pl.ds(base + g * _CH, _CH)], sems_out[b])

  in_copy(0, 0).start()

  @pl.loop(0, _NCH, step=2)
  def _(g):
    for b in (0, 1):
      gg = g + b
      nxt = 1 - b

      @pl.when(gg + 1 < _NCH)
      def _():
        in_copy(gg + 1, nxt).start()

      in_copy(gg, b).wait()

      @pl.when(gg >= 2)
      def _():
        out_copy(gg - 2, b).wait()

      xbuf = xbufs[b]
      obuf = obufs[b]

      @plsc.parallel_loop(0, _CH // (_L * _VEC), unroll=2)
      def _(i):
        xs = [xbuf[pl.ds((i * _VEC + k) * _L, _L)] for k in range(_VEC)]
        preds1 = [b1023 < xs[k] for k in range(_VEC)]
        poss = [jnp.where(preds1[k], lane_hi, lane) for k in range(_VEC)]
        mv2s = [jnp.where(preds1[k], b1535, b511) for k in range(_VEC)]
        poss = [jnp.where(mv2s[k] < xs[k], poss[k] + _L * 512, poss[k])
                for k in range(_VEC)]
        for tab, off, w in ((s128, 2048, 256), (s128, 0, 128), (s64h, 0, 64)):
          ts = [(poss[k] + off if off else poss[k]) >> 11 for k in range(_VEC)]
          mvs = [jnp.take_along_axis(tab, ts[k], axis=0,
                                     mode="promise_in_bounds")
                 for k in range(_VEC)]
          poss = [jnp.where(mvs[k] < xs[k], poss[k] + _L * w, poss[k])
                  for k in range(_VEC)]
        for w in (32, 16, 8, 4, 2, 1):
          idxs = [poss[k] + _L * w for k in range(_VEC)]
          mvs = [plsc.load_gather(means16_v, [idxs[k]]) for k in range(_VEC)]
          poss = [jnp.where(mvs[k] < xs[k], idxs[k], poss[k])
                  for k in range(_VEC)]
        for k in range(_VEC):
          xv = xs[k]
          poslc = jnp.minimum(jnp.maximum(poss[k], lob), hib)
          m1 = plsc.load_gather(means16_v, [poslc])
          pw = plsc.load_gather(pair16_v, [poslc])
          c2 = lax.bitcast_convert_type(
              pw & jnp.int32(-65536), jnp.float32)
          sl = lax.bitcast_convert_type(pw << 16, jnp.float32)
          wa = c2 - (xv - m1) * sl
          res = jnp.where(xv <= m0, 0.0, wa)
          res = jnp.where(xv >= mn, 1.0, res)
          obuf[pl.ds((i * _VEC + k) * _L, _L)] = res

      out_copy(gg, b).start()

  out_copy(_NCH - 2, 0).wait()
  out_copy(_NCH - 1, 1).wait()


@functools.cache
def _build():
  return pl.kernel(
      _body,
      out_type=jax.ShapeDtypeStruct((_N,), jnp.float32),
      mesh=plsc.VectorSubcoreMesh(core_axis_name="c", subcore_axis_name="s",
                                  num_cores=_NC, num_subcores=_NS),
      scratch_types=[
          pltpu.VMEM(((_PAD + 1) * _L,), jnp.float32),
          pltpu.VMEM(((_NTAB + 1) * _L,), jnp.int32),
          pltpu.VMEM((_CH,), jnp.float32),
          pltpu.VMEM((_CH,), jnp.float32),
          pltpu.VMEM((_CH,), jnp.float32),
          pltpu.VMEM((_CH,), jnp.float32),
          pltpu.SemaphoreType.DMA,
          pltpu.SemaphoreType.DMA,
          pltpu.SemaphoreType.DMA,
          pltpu.SemaphoreType.DMA,
      ],
      compiler_params=pltpu.CompilerParams(needs_layout_passes=False),
  )


def kernel(x, processed_means, processed_weights):
  return _build()(x, processed_means, processed_weights)


# speedup vs baseline: 1.2099x; 1.2099x over previous
import functools

import jax
import jax.numpy as jnp
from jax import lax
from jax.experimental import pallas as pl
from jax.experimental.pallas import tpu as pltpu
from jax.experimental.pallas import tpu_sc as plsc

_N = 8388608
_NTAB = 2000
_PAD = 2048
_L = 16
_NC, _NS = 2, 16
_NW = _NC * _NS
_PER_W = _N // _NW
_CH = 8192
_NCH = _PER_W // _CH
_VEC = 4


def _body(x_hbm, means_hbm, weights_hbm, out_hbm,
          means16_v, pair16_v, xb0, xb1, ob0, ob1,
          sem_in0, sem_in1, sem_out0, sem_out1):
  wid = lax.axis_index("s") * _NC + lax.axis_index("c")
  base = wid * _PER_W

  means_v = ob1
  cw_v = ob0
  pltpu.sync_copy(means_hbm, means_v.at[pl.ds(0, _NTAB)])
  pltpu.sync_copy(weights_hbm, cw_v.at[pl.ds(0, _NTAB)])
  for j in range(_NTAB, _PAD, _L):
    means_v[pl.ds(j, _L)] = jnp.full((_L,), jnp.inf, jnp.float32)

  def cw_step(j, carry):
    v = cw_v[pl.ds(j * _L, _L)]
    cw_v[pl.ds(j * _L, _L)] = plsc.cumsum(v) - v * 0.5 + carry
    return carry + jnp.sum(v)

  total_w = lax.fori_loop(0, _NTAB // _L, cw_step, jnp.float32(0.0))
  inv_w = jnp.ones((_L,), jnp.float32) / jnp.broadcast_to(total_w, (_L,))
  m0 = means_v[pl.ds(0, _L)][0]
  mn = means_v[pl.ds(_NTAB - _L, _L)][_L - 1]

  def scale_step(j, _):
    cw_v[pl.ds(j * _L, _L)] = cw_v[pl.ds(j * _L, _L)] * inv_w
    return _

  lax.fori_loop(0, _NTAB // _L, scale_step, jnp.float32(0.0))
  cw_v[pl.ds(_NTAB, _L)] = jnp.ones((_L,), jnp.float32)

  def rep_step(jj, _):
    v = means_v[pl.ds(jj * _L, _L)]
    for k in range(_L):
      means16_v[pl.ds((jj * _L + k + 1) * _L, _L)] = jnp.broadcast_to(v[k], (_L,))
    return _

  lax.fori_loop(0, _PAD // _L, rep_step, jnp.float32(0.0))

  def tab_step(j, _):
    a = cw_v[pl.ds(j * _L, _L)]
    b = cw_v[pl.ds(j * _L + 1, _L)]
    ma = means_v[pl.ds(j * _L, _L)]
    mb = means_v[pl.ds(j * _L + 1, _L)]
    gap = mb - ma
    sl = jnp.where(gap > 0.0, (b - a) / jnp.where(gap > 0.0, gap, 1.0), 0.0)
    bb = lax.bitcast_convert_type(b, jnp.uint32)
    bs = lax.bitcast_convert_type(sl, jnp.uint32)
    word = lax.bitcast_convert_type(
        ((bb + 0x8000) & jnp.uint32(0xFFFF0000)) | ((bs + 0x8000) >> 16),
        jnp.int32)
    for k in range(_L):
      pair16_v[pl.ds((j * _L + k + 1) * _L, _L)] = jnp.broadcast_to(word[k], (_L,))
    return _

  lax.fori_loop(0, _NTAB // _L, tab_step, jnp.float32(0.0))

  lane = lax.iota(jnp.int32, _L)
  lane_hi = lane + _L * 1024
  lob = lane + _L
  hib = lane + _L * (_NTAB - 1)
  b1023 = jnp.broadcast_to(means_v[pl.ds(1008, _L)][_L - 1], (_L,))
  b511 = jnp.broadcast_to(means_v[pl.ds(496, _L)][_L - 1], (_L,))
  b1535 = jnp.broadcast_to(means_v[pl.ds(1520, _L)][_L - 1], (_L,))

  xbufs = (xb0, xb1)
  obufs = (ob0, ob1)
  sems_in = (sem_in0, sem_in1)
  sems_out = (sem_out0, sem_out1)

  def in_copy(g, b):
    return pltpu.make_async_copy(
        x_hbm.at[pl.ds(base + g * _CH, _CH)], xbufs[b], sems_in[b])

  def out_copy(g, b):
    return pltpu.make_async_copy(
        obufs[b], out_hbm.at[pl.ds(base + g * _CH, _CH)], sems_out[b])

  in_copy(0, 0).start()

  @pl.loop(0, _NCH, step=2)
  def _(g):
    for b in (0, 1):
      gg = g + b
      nxt = 1 - b

      @pl.when(gg + 1 < _NCH)
      def _():
        in_copy(gg + 1, nxt).start()

      in_copy(gg, b).wait()

      @pl.when(gg >= 2)
      def _():
        out_copy(gg - 2, b).wait()

      xbuf = xbufs[b]
      obuf = obufs[b]

      @plsc.parallel_loop(0, _CH // (_L * _VEC), unroll=2)
      def _(i):
        xs = [xbuf[pl.ds((i * _VEC + k) * _L, _L)] for k in range(_VEC)]
        preds1 = [b1023 < xs[k] for k in range(_VEC)]
        poss = [jnp.where(preds1[k], lane_hi, lane) for k in range(_VEC)]
        mv2s = [jnp.where(preds1[k], b1535, b511) for k in range(_VEC)]
        poss = [jnp.where(mv2s[k] < xs[k], poss[k] + _L * 512, poss[k])
                for k in range(_VEC)]
        for w in (256, 128, 64, 32, 16, 8, 4, 2, 1):
          idxs = [poss[k] + _L * w for k in range(_VEC)]
          mvs = [plsc.load_gather(means16_v, [idxs[k]]) for k in range(_VEC)]
          poss = [jnp.where(mvs[k] < xs[k], idxs[k], poss[k])
                  for k in range(_VEC)]
        for k in range(_VEC):
          xv = xs[k]
          poslc = jnp.minimum(jnp.maximum(poss[k], lob), hib)
          m1 = plsc.load_gather(means16_v, [poslc])
          pw = plsc.load_gather(pair16_v, [poslc])
          c2 = lax.bitcast_convert_type(
              pw & jnp.int32(-65536), jnp.float32)
          sl = lax.bitcast_convert_type(pw << 16, jnp.float32)
          wa = c2 - (xv - m1) * sl
          res = jnp.where(xv <= m0, 0.0, wa)
          res = jnp.where(xv >= mn, 1.0, res)
          obuf[pl.ds((i * _VEC + k) * _L, _L)] = res

      out_copy(gg, b).start()

  out_copy(_NCH - 2, 0).wait()
  out_copy(_NCH - 1, 1).wait()


@functools.cache
def _build():
  return pl.kernel(
      _body,
      out_type=jax.ShapeDtypeStruct((_N,), jnp.float32),
      mesh=plsc.VectorSubcoreMesh(core_axis_name="c", subcore_axis_name="s",
                                  num_cores=_NC, num_subcores=_NS),
      scratch_types=[
          pltpu.VMEM(((_PAD + 1) * _L,), jnp.float32),
          pltpu.VMEM(((_NTAB + 1) * _L,), jnp.int32),
          pltpu.VMEM((_CH,), jnp.float32),
          pltpu.VMEM((_CH,), jnp.float32),
          pltpu.VMEM((_CH,), jnp.float32),
          pltpu.VMEM((_CH,), jnp.float32),
          pltpu.SemaphoreType.DMA,
          pltpu.SemaphoreType.DMA,
          pltpu.SemaphoreType.DMA,
          pltpu.SemaphoreType.DMA,
      ],
      compiler_params=pltpu.CompilerParams(needs_layout_passes=False),
  )


def kernel(x, processed_means, processed_weights):
  return _build()(x, processed_means, processed_weights)


# speedup vs baseline: 1.2263x; 1.0135x over previous
import functools

import jax
import jax.numpy as jnp
from jax import lax
from jax.experimental import pallas as pl
from jax.experimental.pallas import tpu as pltpu
from jax.experimental.pallas import tpu_sc as plsc

_N = 8388608
_NTAB = 2000
_PAD = 2048
_L = 16
_NC, _NS = 2, 16
_NW = _NC * _NS
_PER_W = _N // _NW
_CH = 8192
_NCH = _PER_W // _CH
_VEC = 4


def _body(x_hbm, means_hbm, weights_hbm, out_hbm,
          means16_v, cw16_v, sl16_v, xb0, xb1, ob0, ob1,
          sem_in0, sem_in1, sem_out0, sem_out1):
  wid = lax.axis_index("s") * _NC + lax.axis_index("c")
  base = wid * _PER_W

  means_v = ob1
  cw_v = ob0
  pltpu.sync_copy(means_hbm, means_v.at[pl.ds(0, _NTAB)])
  pltpu.sync_copy(weights_hbm, cw_v.at[pl.ds(0, _NTAB)])
  for j in range(_NTAB, _PAD, _L):
    means_v[pl.ds(j, _L)] = jnp.full((_L,), jnp.inf, jnp.float32)

  def cw_step(j, carry):
    v = cw_v[pl.ds(j * _L, _L)]
    cw_v[pl.ds(j * _L, _L)] = plsc.cumsum(v) - v * 0.5 + carry
    return carry + jnp.sum(v)

  total_w = lax.fori_loop(0, _NTAB // _L, cw_step, jnp.float32(0.0))
  inv_w = jnp.ones((_L,), jnp.float32) / jnp.broadcast_to(total_w, (_L,))
  m0 = means_v[pl.ds(0, _L)][0]
  mn = means_v[pl.ds(_NTAB - _L, _L)][_L - 1]

  def scale_step(j, _):
    cw_v[pl.ds(j * _L, _L)] = cw_v[pl.ds(j * _L, _L)] * inv_w
    return _

  lax.fori_loop(0, _NTAB // _L, scale_step, jnp.float32(0.0))
  cw_v[pl.ds(_NTAB, _L)] = jnp.ones((_L,), jnp.float32)

  def rep_step(jj, _):
    v = means_v[pl.ds(jj * _L, _L)]
    for k in range(_L):
      means16_v[pl.ds((jj * _L + k + 1) * _L, _L)] = jnp.broadcast_to(v[k], (_L,))
    return _

  lax.fori_loop(0, _PAD // _L, rep_step, jnp.float32(0.0))

  def tab_step(j, _):
    a = cw_v[pl.ds(j * _L, _L)]
    b = cw_v[pl.ds(j * _L + 1, _L)]
    ma = means_v[pl.ds(j * _L, _L)]
    mb = means_v[pl.ds(j * _L + 1, _L)]
    gap = mb - ma
    sl = jnp.where(gap > 0.0, (b - a) / jnp.where(gap > 0.0, gap, 1.0), 0.0)
    for k in range(_L):
      cw16_v[pl.ds((j * _L + k) * _L, _L)] = jnp.broadcast_to(a[k], (_L,))
      sl16_v[pl.ds((j * _L + k + 1) * _L, _L)] = jnp.broadcast_to(sl[k], (_L,))
    return _

  lax.fori_loop(0, _NTAB // _L, tab_step, jnp.float32(0.0))

  lane = lax.iota(jnp.int32, _L)
  lane_hi = lane + _L * 1024
  lob = lane + _L
  hib = lane + _L * (_NTAB - 1)
  b1023 = jnp.broadcast_to(means_v[pl.ds(1008, _L)][_L - 1], (_L,))
  b511 = jnp.broadcast_to(means_v[pl.ds(496, _L)][_L - 1], (_L,))
  b1535 = jnp.broadcast_to(means_v[pl.ds(1520, _L)][_L - 1], (_L,))

  xbufs = (xb0, xb1)
  obufs = (ob0, ob1)
  sems_in = (sem_in0, sem_in1)
  sems_out = (sem_out0, sem_out1)

  def in_copy(g, b):
    return pltpu.make_async_copy(
        x_hbm.at[pl.ds(base + g * _CH, _CH)], xbufs[b], sems_in[b])

  def out_copy(g, b):
    return pltpu.make_async_copy(
        obufs[b], out_hbm.at[pl.ds(base + g * _CH, _CH)], sems_out[b])

  in_copy(0, 0).start()

  @pl.loop(0, _NCH, step=2)
  def _(g):
    for b in (0, 1):
      gg = g + b
      nxt = 1 - b

      @pl.when(gg + 1 < _NCH)
      def _():
        in_copy(gg + 1, nxt).start()

      in_copy(gg, b).wait()

      @pl.when(gg >= 2)
      def _():
        out_copy(gg - 2, b).wait()

      xbuf = xbufs[b]
      obuf = obufs[b]

      @plsc.parallel_loop(0, _CH // (_L * _VEC), unroll=2)
      def _(i):
        xs = [xbuf[pl.ds((i * _VEC + k) * _L, _L)] for k in range(_VEC)]
        preds1 = [b1023 < xs[k] for k in range(_VEC)]
        poss = [jnp.where(preds1[k], lane_hi, lane) for k in range(_VEC)]
        mv2s = [jnp.where(preds1[k], b1535, b511) for k in range(_VEC)]
        poss = [jnp.where(mv2s[k] < xs[k], poss[k] + _L * 512, poss[k])
                for k in range(_VEC)]
        for w in (256, 128, 64, 32, 16, 8, 4, 2, 1):
          idxs = [poss[k] + _L * w for k in range(_VEC)]
          mvs = [plsc.load_gather(means16_v, [idxs[k]]) for k in range(_VEC)]
          poss = [jnp.where(mvs[k] < xs[k], idxs[k], poss[k])
                  for k in range(_VEC)]
        for k in range(_VEC):
          xv = xs[k]
          poslc = jnp.minimum(jnp.maximum(poss[k], lob), hib)
          m1 = plsc.load_gather(means16_v, [poslc])
          c2 = plsc.load_gather(cw16_v, [poslc])
          sl = plsc.load_gather(sl16_v, [poslc])
          wa = c2 - (xv - m1) * sl
          res = jnp.where(xv <= m0, 0.0, wa)
          res = jnp.where(xv >= mn, 1.0, res)
          obuf[pl.ds((i * _VEC + k) * _L, _L)] = res

      out_copy(gg, b).start()

  out_copy(_NCH - 2, 0).wait()
  out_copy(_NCH - 1, 1).wait()


@functools.cache
def _build():
  return pl.kernel(
      _body,
      out_type=jax.ShapeDtypeStruct((_N,), jnp.float32),
      mesh=plsc.VectorSubcoreMesh(core_axis_name="c", subcore_axis_name="s",
                                  num_cores=_NC, num_subcores=_NS),
      scratch_types=[
          pltpu.VMEM(((_PAD + 1) * _L,), jnp.float32),
          pltpu.VMEM((_NTAB * _L,), jnp.float32),
          pltpu.VMEM(((_NTAB + 1) * _L,), jnp.float32),
          pltpu.VMEM((_CH,), jnp.float32),
          pltpu.VMEM((_CH,), jnp.float32),
          pltpu.VMEM((_CH,), jnp.float32),
          pltpu.VMEM((_CH,), jnp.float32),
          pltpu.SemaphoreType.DMA,
          pltpu.SemaphoreType.DMA,
          pltpu.SemaphoreType.DMA,
          pltpu.SemaphoreType.DMA,
      ],
      compiler_params=pltpu.CompilerParams(needs_layout_passes=False),
  )


def kernel(x, processed_means, processed_weights):
  return _build()(x, processed_means, processed_weights)
